# baseline (speedup 1.0000x reference)
import functools

import jax
import jax.numpy as jnp
from jax import lax
from jax.experimental import pallas as pl
from jax.experimental.pallas import tpu as pltpu
from jax.experimental.pallas import tpu_sc as plsc

_N = 10000
_E = 320000
_B = 16
_DH = 128
_NL = 3
_VOCAB = 16
_T = 1000
_PREC = 1e-05

_NP = 10240
_EP = 327680
_NWORK = 32
_CHUNK = 128
_NCH = _EP // (_NWORK * _CHUNK)
_XW = 128
_NR = _NP // 16
_BLKN = 1024
_BLKE = 2048
_XSCALE = 1.0 / (1.0 + _E / _N)


def _silu(z):
    return z * lax.logistic(z)


def _sc_gather_body(a_hbm, bt_hbm, src_hbm, dst_hbm, g1_hbm, g2_hbm,
                    idx_s, idx_d, b1_0, b1_1, b2_0, b2_1,
                    gsem0, gsem1, wsem0, wsem1):
    wid = lax.axis_index("s") * 2 + lax.axis_index("c")
    bufs1 = (b1_0, b1_1)
    bufs2 = (b2_0, b2_1)
    gsems = (gsem0, gsem1)
    wsems = (wsem0, wsem1)

    pltpu.sync_copy(src_hbm.at[wid], idx_s)
    pltpu.sync_copy(dst_hbm.at[wid], idx_d)

    def g_start(j, k):
        pltpu.async_copy(a_hbm.at[idx_s.at[j]], bufs1[k], gsems[k])
        pltpu.async_copy(bt_hbm.at[idx_d.at[j]], bufs2[k], gsems[k])

    def g_wait(k):
        pltpu.make_async_copy(a_hbm.at[idx_s.at[0]], bufs1[k], gsems[k]).wait()
        pltpu.make_async_copy(a_hbm.at[idx_s.at[0]], bufs2[k], gsems[k]).wait()

    def w_start(j, k):
        base = (wid * _NCH + j) * _CHUNK
        pltpu.async_copy(bufs1[k], g1_hbm.at[pl.ds(base, _CHUNK)], wsems[k])
        pltpu.async_copy(bufs2[k], g2_hbm.at[pl.ds(base, _CHUNK)], wsems[k])

    def w_wait(k):
        pltpu.make_async_copy(bufs1[k], g1_hbm.at[pl.ds(0, _CHUNK)],
                              wsems[k]).wait()
        pltpu.make_async_copy(bufs2[k], g2_hbm.at[pl.ds(0, _CHUNK)],
                              wsems[k]).wait()

    def body(t, carry):
        j0 = 2 * t
        j1 = j0 + 1

        @pl.when(t >= 1)
        def _():
            w_wait(0)

        g_start(j0, 0)

        @pl.when(t >= 1)
        def _():
            g_wait(1)
            w_start(j0 - 1, 1)
            w_wait(1)

        g_start(j1, 1)
        g_wait(0)
        w_start(j0, 0)
        return carry

    lax.fori_loop(0, _NCH // 2, body, 0)
    g_wait(1)
    w_start(_NCH - 1, 1)
    w_wait(0)
    w_wait(1)


def _sc_scatter_body(s_hbm, dst_hbm, p_hbm, idx_0, idx_1, b_0, b_1, acc,
                     lsem0, lsem1, csem0, csem1):
    cid = lax.axis_index("c")
    sid = lax.axis_index("s")
    wid = sid * 2 + cid
    idxs = (idx_0, idx_1)
    bufs = (b_0, b_1)
    lsems = (lsem0, lsem1)
    csems = (csem0, csem1)
    ncol = b_0.shape[1]

    def zrow(i, carry):
        for k in range(ncol // 16):
            b_0[i, pl.ds(k * 16, 16)] = jnp.zeros((16,), jnp.float32)
        return carry

    lax.fori_loop(0, _CHUNK, zrow, 0)

    def zacc(t, carry):
        pltpu.sync_copy(b_0, acc.at[pl.ds(sid * _NR + t * _CHUNK, _CHUNK)])
        return carry

    lax.fori_loop(0, _NR // _CHUNK, zacc, 0)
    plsc.subcore_barrier()

    def l_start(j, k):
        base = (wid * _NCH + j) * _CHUNK
        pltpu.async_copy(s_hbm.at[pl.ds(base, _CHUNK)], bufs[k], lsems[k])
        pltpu.async_copy(dst_hbm.at[wid, j], idxs[k], lsems[k])

    def l_wait(k):
        pltpu.make_async_copy(s_hbm.at[pl.ds(0, _CHUNK)], bufs[k],
                              lsems[k]).wait()
        pltpu.make_async_copy(dst_hbm.at[0, 0], idxs[k], lsems[k]).wait()

    def c_start(j, k):
        pltpu.async_copy(bufs[k], acc.at[idxs[k]], csems[k], add=True)

    def c_wait(k):
        pltpu.make_async_copy(bufs[k], acc.at[idxs[k]], csems[k]).wait()

    def body(t, carry):
        j0 = 2 * t
        j1 = j0 + 1

        @pl.when(t >= 1)
        def _():
            l_wait(1)
            c_start(j0 - 1, 1)
            c_wait(0)

        l_start(j0, 0)
        l_wait(0)

        @pl.when(t >= 1)
        def _():
            c_wait(1)

        c_start(j0, 0)
        l_start(j1, 1)
        return carry

    lax.fori_loop(0, _NCH // 2, body, 0)
    l_wait(1)
    c_start(_NCH - 1, 1)
    c_wait(0)
    c_wait(1)
    plsc.subcore_barrier()
    pltpu.sync_copy(acc.at[pl.ds(sid * _NR, _NR)],
                    p_hbm.at[cid, pl.ds(sid * _NR, _NR)])


def _make_gather(width, tiled):
    mesh = plsc.VectorSubcoreMesh(core_axis_name="c", subcore_axis_name="s",
                                  num_cores=2)
    return pl.kernel(
        _sc_gather_body,
        out_type=(
            jax.ShapeDtypeStruct((_EP, width), jnp.float32),
            jax.ShapeDtypeStruct((_EP, width), jnp.float32),
        ),
        mesh=mesh,
        scratch_types=[
            pltpu.VMEM((_NCH, _CHUNK), jnp.int32),
            pltpu.VMEM((_NCH, _CHUNK), jnp.int32),
            pltpu.VMEM((_CHUNK, width), jnp.float32),
            pltpu.VMEM((_CHUNK, width), jnp.float32),
            pltpu.VMEM((_CHUNK, width), jnp.float32),
            pltpu.VMEM((_CHUNK, width), jnp.float32),
            pltpu.SemaphoreType.DMA,
            pltpu.SemaphoreType.DMA,
            pltpu.SemaphoreType.DMA,
            pltpu.SemaphoreType.DMA,
        ],
        compiler_params=pltpu.CompilerParams(use_tc_tiling_on_sc=tiled),
    )


def _make_scatter(width, tiled):
    mesh = plsc.VectorSubcoreMesh(core_axis_name="c", subcore_axis_name="s",
                                  num_cores=2)
    return pl.kernel(
        _sc_scatter_body,
        out_type=jax.ShapeDtypeStruct((2, _NP, width), jnp.float32),
        mesh=mesh,
        scratch_types=[
            pltpu.VMEM((_CHUNK,), jnp.int32),
            pltpu.VMEM((_CHUNK,), jnp.int32),
            pltpu.VMEM((_CHUNK, width), jnp.float32),
            pltpu.VMEM((_CHUNK, width), jnp.float32),
            pltpu.VMEM_SHARED((_NP, width), jnp.float32),
            pltpu.SemaphoreType.DMA,
            pltpu.SemaphoreType.DMA,
            pltpu.SemaphoreType.DMA,
            pltpu.SemaphoreType.DMA,
        ],
        compiler_params=pltpu.CompilerParams(use_tc_tiling_on_sc=tiled),
    )


@functools.cache
def _sc_kernels():
    return (_make_gather(128, True), _make_gather(_XW, True),
            _make_scatter(128, True), _make_scatter(_XW, True))


def _sc_gather_big(a, bt, src3, dst3):
    return _sc_kernels()[0](a, bt, src3, dst3)


def _sc_gather_x(xp, src3, dst3):
    return _sc_kernels()[1](xp, xp, src3, dst3)


def _sc_scatter_big(s, dst3):
    return _sc_kernels()[2](s, dst3)


def _sc_scatter_x(sx, dst3):
    return _sc_kernels()[3](sx, dst3)


def _onehot16(ids_col, rows):
    return (ids_col == lax.broadcasted_iota(jnp.int32, (rows, 16), 1)
            ).astype(jnp.float32)


def _segsum0_body(eps_ref, gid_ref, out_ref):
    @pl.when(pl.program_id(0) == 0)
    def _():
        out_ref[...] = jnp.zeros_like(out_ref)

    oh = _onehot16(gid_ref[...], _BLKN)
    vals = jnp.concatenate(
        [eps_ref[:, :16], jnp.ones((_BLKN, 16), jnp.float32)], axis=1)
    out_ref[...] += lax.dot_general(
        oh, vals, (((0,), (0,)), ((), ())),
        preferred_element_type=jnp.float32)


def _segsum0(eps_p, gid_p):
    grid = _NP // _BLKN
    return pl.pallas_call(
        _segsum0_body,
        grid=(grid,),
        in_specs=[
            pl.BlockSpec((_BLKN, _XW), lambda i: (i, 0)),
            pl.BlockSpec((_BLKN, 1), lambda i: (i, 0)),
        ],
        out_specs=pl.BlockSpec((16, 32), lambda i: (0, 0)),
        out_shape=jax.ShapeDtypeStruct((16, 32), jnp.float32),
    )(eps_p, gid_p)


def _node_init_body(xyz_ref, eps_ref, aid_ref, gid_ref, gtab_ref, gmean_ref,
                    wemb_ref, wembt_ref, bemb_ref, h_ref, x_ref, epsc_ref):
    oh_g = _onehot16(gid_ref[...], _BLKN)
    per = jnp.dot(oh_g, gtab_ref[...], preferred_element_type=jnp.float32)
    meanp = jnp.dot(oh_g, gmean_ref[...], preferred_element_type=jnp.float32)
    alpha = per[:, 0:1]
    sigma = per[:, 1:2]
    tn = per[:, 2:3]
    epsc = eps_ref[...] - meanp
    x0 = alpha * xyz_ref[...] + sigma * epsc
    oh_a = _onehot16(aid_ref[...], _BLKN)
    h0 = _silu(jnp.dot(oh_a, wemb_ref[...], preferred_element_type=jnp.float32)
               + tn * wembt_ref[...] + bemb_ref[...])
    h_ref[...] = h0
    x_ref[...] = x0
    epsc_ref[...] = epsc


def _node_init(xyz_p, eps_p, aid_p, gid_p, gtab, gmean, wemb, wembt, bemb):
    grid = _NP // _BLKN
    return pl.pallas_call(
        _node_init_body,
        grid=(grid,),
        in_specs=[
            pl.BlockSpec((_BLKN, _XW), lambda i: (i, 0)),
            pl.BlockSpec((_BLKN, _XW), lambda i: (i, 0)),
            pl.BlockSpec((_BLKN, 1), lambda i: (i, 0)),
            pl.BlockSpec((_BLKN, 1), lambda i: (i, 0)),
            pl.BlockSpec((16, 8), lambda i: (0, 0)),
            pl.BlockSpec((16, _XW), lambda i: (0, 0)),
            pl.BlockSpec((16, 128), lambda i: (0, 0)),
            pl.BlockSpec((1, 128), lambda i: (0, 0)),
            pl.BlockSpec((1, 128), lambda i: (0, 0)),
        ],
        out_specs=[
            pl.BlockSpec((_BLKN, 128), lambda i: (i, 0)),
            pl.BlockSpec((_BLKN, _XW), lambda i: (i, 0)),
            pl.BlockSpec((_BLKN, _XW), lambda i: (i, 0)),
        ],
        out_shape=[
            jax.ShapeDtypeStruct((_NP, 128), jnp.float32),
            jax.ShapeDtypeStruct((_NP, _XW), jnp.float32),
            jax.ShapeDtypeStruct((_NP, _XW), jnp.float32),
        ],
    )(xyz_p, eps_p, aid_p, gid_p, gtab, gmean, wemb, wembt, bemb)


def _pre_body(h_ref, w1a_ref, w1b_ref, b1_ref, a_ref, bt_ref):
    h = h_ref[...]
    a_ref[...] = jnp.dot(h, w1a_ref[...],
                         preferred_element_type=jnp.float32) + b1_ref[...]
    bt_ref[...] = jnp.dot(h, w1b_ref[...],
                          preferred_element_type=jnp.float32)


def _pre(h, w1a, w1b, b1):
    grid = _NP // _BLKN
    return pl.pallas_call(
        _pre_body,
        grid=(grid,),
        in_specs=[
            pl.BlockSpec((_BLKN, 128), lambda i: (i, 0)),
            pl.BlockSpec((128, 128), lambda i: (0, 0)),
            pl.BlockSpec((128, 128), lambda i: (0, 0)),
            pl.BlockSpec((1, 128), lambda i: (0, 0)),
        ],
        out_specs=[
            pl.BlockSpec((_BLKN, 128), lambda i: (i, 0)),
            pl.BlockSpec((_BLKN, 128), lambda i: (i, 0)),
        ],
        out_shape=[
            jax.ShapeDtypeStruct((_NP, 128), jnp.float32),
            jax.ShapeDtypeStruct((_NP, 128), jnp.float32),
        ],
    )(h, w1a, w1b, b1)


def _edge_body(g1_ref, g2_ref, gx1_ref, gx2_ref, w1c_ref, w2_ref, b2_ref,
               wx_ref, s_ref, sx_ref):
    pre = g1_ref[...] + g2_ref[...]
    diff = gx1_ref[...] - gx2_ref[...]
    d2 = jnp.sum(diff * diff, axis=1, keepdims=True)
    m1 = _silu(pre + d2 * w1c_ref[...])
    m2 = _silu(jnp.dot(m1, w2_ref[...], preferred_element_type=jnp.float32)
               + b2_ref[...])
    coef = jnp.tanh(jnp.sum(m2 * wx_ref[...], axis=1, keepdims=True))
    s_ref[...] = m2
    sx_ref[...] = diff * coef


def _edge(g1, g2, gx1, gx2, w1c, w2, b2, wx):
    grid = _EP // _BLKE
    return pl.pallas_call(
        _edge_body,
        grid=(grid,),
        in_specs=[
            pl.BlockSpec((_BLKE, 128), lambda i: (i, 0)),
            pl.BlockSpec((_BLKE, 128), lambda i: (i, 0)),
            pl.BlockSpec((_BLKE, _XW), lambda i: (i, 0)),
            pl.BlockSpec((_BLKE, _XW), lambda i: (i, 0)),
            pl.BlockSpec((1, 128), lambda i: (0, 0)),
            pl.BlockSpec((128, 128), lambda i: (0, 0)),
            pl.BlockSpec((1, 128), lambda i: (0, 0)),
            pl.BlockSpec((1, 128), lambda i: (0, 0)),
        ],
        out_specs=[
            pl.BlockSpec((_BLKE, 128), lambda i: (i, 0)),
            pl.BlockSpec((_BLKE, _XW), lambda i: (i, 0)),
        ],
        out_shape=[
            jax.ShapeDtypeStruct((_EP, 128), jnp.float32),
            jax.ShapeDtypeStruct((_EP, _XW), jnp.float32),
        ],
    )(g1, g2, gx1, gx2, w1c, w2, b2, wx)


def _update_body(h_ref, x_ref, p_ref, px_ref, wuh_ref, wua_ref, bu_ref,
                 h2_ref, x2_ref):
    agg = p_ref[0] + p_ref[1]
    aggx = px_ref[0] + px_ref[1]
    h = h_ref[...]
    z = (jnp.dot(h, wuh_ref[...], preferred_element_type=jnp.float32)
         + jnp.dot(agg, wua_ref[...], preferred_element_type=jnp.float32)
         + bu_ref[...])
    h2_ref[...] = h + _silu(z)
    x2_ref[...] = x_ref[...] + aggx * _XSCALE


def _update(h, x, p, px, wuh, wua, bu):
    grid = _NP // _BLKN
    return pl.pallas_call(
        _update_body,
        grid=(grid,),
        in_specs=[
            pl.BlockSpec((_BLKN, 128), lambda i: (i, 0)),
            pl.BlockSpec((_BLKN, _XW), lambda i: (i, 0)),
            pl.BlockSpec((2, _BLKN, 128), lambda i: (0, i, 0)),
            pl.BlockSpec((2, _BLKN, _XW), lambda i: (0, i, 0)),
            pl.BlockSpec((128, 128), lambda i: (0, 0)),
            pl.BlockSpec((128, 128), lambda i: (0, 0)),
            pl.BlockSpec((1, 128), lambda i: (0, 0)),
        ],
        out_specs=[
            pl.BlockSpec((_BLKN, 128), lambda i: (i, 0)),
            pl.BlockSpec((_BLKN, _XW), lambda i: (i, 0)),
        ],
        out_shape=[
            jax.ShapeDtypeStruct((_NP, 128), jnp.float32),
            jax.ShapeDtypeStruct((_NP, _XW), jnp.float32),
        ],
    )(h, x, p, px, wuh, wua, bu)


def _velsum_body(x3_ref, x0_ref, gid_ref, out_ref):
    @pl.when(pl.program_id(0) == 0)
    def _():
        out_ref[...] = jnp.zeros_like(out_ref)

    oh = _onehot16(gid_ref[...], _BLKN)
    vel = x3_ref[...] - x0_ref[...]
    out_ref[...] += lax.dot_general(
        oh, vel, (((0,), (0,)), ((), ())),
        preferred_element_type=jnp.float32)


def _velsum(x3, x0, gid_p):
    grid = _NP // _BLKN
    return pl.pallas_call(
        _velsum_body,
        grid=(grid,),
        in_specs=[
            pl.BlockSpec((_BLKN, _XW), lambda i: (i, 0)),
            pl.BlockSpec((_BLKN, _XW), lambda i: (i, 0)),
            pl.BlockSpec((_BLKN, 1), lambda i: (i, 0)),
        ],
        out_specs=pl.BlockSpec((16, _XW), lambda i: (0, 0)),
        out_shape=jax.ShapeDtypeStruct((16, _XW), jnp.float32),
    )(x3, x0, gid_p)


def _errsum_body(x3_ref, x0_ref, epsc_ref, gid_ref, mv_ref, out_ref):
    @pl.when(pl.program_id(0) == 0)
    def _():
        out_ref[...] = jnp.zeros_like(out_ref)

    oh = _onehot16(gid_ref[...], _BLKN)
    velc = (x3_ref[...] - x0_ref[...]
            - jnp.dot(oh, mv_ref[...], preferred_element_type=jnp.float32))
    err = (velc - epsc_ref[...]) ** 2
    out_ref[...] += lax.dot_general(
        oh, err, (((0,), (0,)), ((), ())),
        preferred_element_type=jnp.float32)


def _errsum(x3, x0, epsc, gid_p, mv):
    grid = _NP // _BLKN
    return pl.pallas_call(
        _errsum_body,
        grid=(grid,),
        in_specs=[
            pl.BlockSpec((_BLKN, _XW), lambda i: (i, 0)),
            pl.BlockSpec((_BLKN, _XW), lambda i: (i, 0)),
            pl.BlockSpec((_BLKN, _XW), lambda i: (i, 0)),
            pl.BlockSpec((_BLKN, 1), lambda i: (i, 0)),
            pl.BlockSpec((16, _XW), lambda i: (0, 0)),
        ],
        out_specs=pl.BlockSpec((16, _XW), lambda i: (0, 0)),
        out_shape=jax.ShapeDtypeStruct((16, _XW), jnp.float32),
    )(x3, x0, epsc, gid_p, mv)


def kernel(xyz, eps, atom_ids, edge_index, graph_ids, t_int,
           W_emb, b_emb, W_m1, b_m1, W_m2, b_m2, W_u, b_u, W_x):
    f32 = jnp.float32

    xn = t_int.astype(f32) / _T
    a2 = (1.0 - xn ** 2) ** 2
    a2 = (1.0 - 2.0 * _PREC) * a2 + _PREC
    gamma = jnp.log(1.0 - a2) - jnp.log(a2)
    alpha_g = jnp.sqrt(lax.logistic(-gamma))
    sigma_g = jnp.sqrt(lax.logistic(gamma))
    t_g = t_int.astype(f32) / _T

    xyz_p = jnp.zeros((_NP, _XW), f32).at[:_N, :3].set(xyz)
    eps_p = jnp.zeros((_NP, _XW), f32).at[:_N, :3].set(eps)
    aid_p = jnp.zeros((_NP, 1), jnp.int32).at[:_N, 0].set(
        atom_ids.astype(jnp.int32))
    gid_p = jnp.full((_NP, 1), _B, jnp.int32).at[:_N, 0].set(
        graph_ids.astype(jnp.int32))

    src = edge_index[0].astype(jnp.int32)
    dst = edge_index[1].astype(jnp.int32)
    src3 = jnp.zeros((_EP,), jnp.int32).at[:_E].set(src).reshape(
        _NWORK, _NCH, _CHUNK)
    dst3 = jnp.full((_EP,), _N, jnp.int32).at[:_E].set(dst).reshape(
        _NWORK, _NCH, _CHUNK)

    sums = _segsum0(eps_p, gid_p)
    cnt = sums[:, 16:17]
    mean_eps = jnp.zeros((16, _XW), f32).at[:, :16].set(sums[:, :16] / cnt)
    gtab = jnp.zeros((16, 8), f32)
    gtab = gtab.at[:, 0].set(alpha_g).at[:, 1].set(sigma_g).at[:, 2].set(t_g)

    h, x, epsc = _node_init(
        xyz_p, eps_p, aid_p, gid_p, gtab, mean_eps,
        W_emb[:_VOCAB], W_emb[_VOCAB:_VOCAB + 1], b_emb.reshape(1, 128))
    x0 = x

    for l in range(_NL):
        w1a = W_m1[l, :128]
        w1b = W_m1[l, 128:256]
        w1c = W_m1[l, 256:257]
        b1 = b_m1[l].reshape(1, 128)
        w2 = W_m2[l]
        b2 = b_m2[l].reshape(1, 128)
        wx = W_x[l].reshape(1, 128)
        wuh = W_u[l, :128]
        wua = W_u[l, 128:]
        bu = b_u[l].reshape(1, 128)

        a_tab, bt_tab = _pre(h, w1a, w1b, b1)
        g1, g2 = _sc_gather_big(a_tab, bt_tab, src3, dst3)
        gx1, gx2 = _sc_gather_x(x, src3, dst3)
        s_msg, sx_msg = _edge(g1, g2, gx1, gx2, w1c, w2, b2, wx)
        p_acc = _sc_scatter_big(s_msg, dst3)
        px_acc = _sc_scatter_x(sx_msg, dst3)
        h, x = _update(h, x, p_acc, px_acc, wuh, wua, bu)

    vsums = _velsum(x, x0, gid_p)
    mv = vsums / cnt
    esums = _errsum(x, x0, epsc, gid_p, mv)
    per_graph = (esums[:, 0] + esums[:, 1] + esums[:, 2]) / (3.0 * cnt[:, 0])
    return per_graph


# speedup vs baseline: 1.3359x; 1.3359x over previous
import functools

import jax
import jax.numpy as jnp
from jax import lax
from jax.experimental import pallas as pl
from jax.experimental.pallas import tpu as pltpu
from jax.experimental.pallas import tpu_sc as plsc

_N = 10000
_E = 320000
_B = 16
_DH = 128
_NL = 3
_VOCAB = 16
_T = 1000
_PREC = 1e-05

_NP = 10240
_EP = 327680
_NWORK = 32
_CHUNK = 128
_NCH = _EP // (_NWORK * _CHUNK)
_XW = 16
_NR = _NP // 16
_BLKN = 1024
_BLKE = 2048
_XSCALE = 1.0 / (1.0 + _E / _N)


def _silu(z):
    return z * lax.logistic(z)


def _sc_gather_body(a_hbm, bt_hbm, src_hbm, dst_hbm, g1_hbm, g2_hbm,
                    idx_s, idx_d, b1_0, b1_1, b2_0, b2_1,
                    gsem0, gsem1, wsem0, wsem1):
    wid = lax.axis_index("s") * 2 + lax.axis_index("c")
    bufs1 = (b1_0, b1_1)
    bufs2 = (b2_0, b2_1)
    gsems = (gsem0, gsem1)
    wsems = (wsem0, wsem1)

    pltpu.sync_copy(src_hbm.at[wid], idx_s)
    pltpu.sync_copy(dst_hbm.at[wid], idx_d)

    def g_start(j, k):
        pltpu.async_copy(a_hbm.at[idx_s.at[j]], bufs1[k], gsems[k])
        pltpu.async_copy(bt_hbm.at[idx_d.at[j]], bufs2[k], gsems[k])

    def g_wait(k):
        pltpu.make_async_copy(a_hbm.at[idx_s.at[0]], bufs1[k], gsems[k]).wait()
        pltpu.make_async_copy(a_hbm.at[idx_s.at[0]], bufs2[k], gsems[k]).wait()

    def w_start(j, k):
        base = (wid * _NCH + j) * _CHUNK
        pltpu.async_copy(bufs1[k], g1_hbm.at[pl.ds(base, _CHUNK)], wsems[k])
        pltpu.async_copy(bufs2[k], g2_hbm.at[pl.ds(base, _CHUNK)], wsems[k])

    def w_wait(k):
        pltpu.make_async_copy(bufs1[k], g1_hbm.at[pl.ds(0, _CHUNK)],
                              wsems[k]).wait()
        pltpu.make_async_copy(bufs2[k], g2_hbm.at[pl.ds(0, _CHUNK)],
                              wsems[k]).wait()

    def body(t, carry):
        j0 = 2 * t
        j1 = j0 + 1

        @pl.when(t >= 1)
        def _():
            w_wait(0)

        g_start(j0, 0)

        @pl.when(t >= 1)
        def _():
            g_wait(1)
            w_start(j0 - 1, 1)
            w_wait(1)

        g_start(j1, 1)
        g_wait(0)
        w_start(j0, 0)
        return carry

    lax.fori_loop(0, _NCH // 2, body, 0)
    g_wait(1)
    w_start(_NCH - 1, 1)
    w_wait(0)
    w_wait(1)


def _sc_scatter_body(s_hbm, dst_hbm, p_hbm, idx_0, idx_1, b_0, b_1, acc,
                     lsem0, lsem1, csem0, csem1):
    cid = lax.axis_index("c")
    sid = lax.axis_index("s")
    wid = sid * 2 + cid
    idxs = (idx_0, idx_1)
    bufs = (b_0, b_1)
    lsems = (lsem0, lsem1)
    csems = (csem0, csem1)
    ncol = b_0.shape[1]

    def zrow(i, carry):
        for k in range(ncol // 16):
            b_0[i, pl.ds(k * 16, 16)] = jnp.zeros((16,), jnp.float32)
        return carry

    lax.fori_loop(0, _CHUNK, zrow, 0)

    def zacc(t, carry):
        pltpu.sync_copy(b_0, acc.at[pl.ds(sid * _NR + t * _CHUNK, _CHUNK)])
        return carry

    lax.fori_loop(0, _NR // _CHUNK, zacc, 0)
    plsc.subcore_barrier()

    def l_start(j, k):
        base = (wid * _NCH + j) * _CHUNK
        pltpu.async_copy(s_hbm.at[pl.ds(base, _CHUNK)], bufs[k], lsems[k])
        pltpu.async_copy(dst_hbm.at[wid, j], idxs[k], lsems[k])

    def l_wait(k):
        pltpu.make_async_copy(s_hbm.at[pl.ds(0, _CHUNK)], bufs[k],
                              lsems[k]).wait()
        pltpu.make_async_copy(dst_hbm.at[0, 0], idxs[k], lsems[k]).wait()

    def c_start(j, k):
        pltpu.async_copy(bufs[k], acc.at[idxs[k]], csems[k], add=True)

    def c_wait(k):
        pltpu.make_async_copy(bufs[k], acc.at[idxs[k]], csems[k]).wait()

    def body(t, carry):
        j0 = 2 * t
        j1 = j0 + 1

        @pl.when(t >= 1)
        def _():
            l_wait(1)
            c_start(j0 - 1, 1)
            c_wait(0)

        l_start(j0, 0)
        l_wait(0)

        @pl.when(t >= 1)
        def _():
            c_wait(1)

        c_start(j0, 0)
        l_start(j1, 1)
        return carry

    lax.fori_loop(0, _NCH // 2, body, 0)
    l_wait(1)
    c_start(_NCH - 1, 1)
    c_wait(0)
    c_wait(1)
    plsc.subcore_barrier()
    pltpu.sync_copy(acc.at[pl.ds(sid * _NR, _NR)],
                    p_hbm.at[cid, pl.ds(sid * _NR, _NR)])


def _sc_gather_diff_body(a_hbm, bt_hbm, src_hbm, dst_hbm, g1_hbm,
                         idx_s, idx_d, b1_0, b1_1, b2_0, b2_1,
                         gsem0, gsem1, wsem0, wsem1):
    wid = lax.axis_index("s") * 2 + lax.axis_index("c")
    bufs1 = (b1_0, b1_1)
    bufs2 = (b2_0, b2_1)
    gsems = (gsem0, gsem1)
    wsems = (wsem0, wsem1)

    pltpu.sync_copy(src_hbm.at[wid], idx_s)
    pltpu.sync_copy(dst_hbm.at[wid], idx_d)

    def g_start(j, k):
        pltpu.async_copy(a_hbm.at[idx_s.at[j]], bufs1[k], gsems[k])
        pltpu.async_copy(bt_hbm.at[idx_d.at[j]], bufs2[k], gsems[k])

    def g_wait(k):
        pltpu.make_async_copy(a_hbm.at[idx_s.at[0]], bufs1[k], gsems[k]).wait()
        pltpu.make_async_copy(a_hbm.at[idx_s.at[0]], bufs2[k], gsems[k]).wait()

    def sub(k):
        def row(i, carry):
            bufs1[k][i, :] = bufs1[k][i, :] - bufs2[k][i, :]
            return carry

        lax.fori_loop(0, _CHUNK, row, 0)

    def w_start(j, k):
        base = (wid * _NCH + j) * _CHUNK
        pltpu.async_copy(bufs1[k], g1_hbm.at[pl.ds(base, _CHUNK)], wsems[k])

    def w_wait(k):
        pltpu.make_async_copy(bufs1[k], g1_hbm.at[pl.ds(0, _CHUNK)],
                              wsems[k]).wait()

    def body(t, carry):
        j0 = 2 * t
        j1 = j0 + 1

        @pl.when(t >= 1)
        def _():
            w_wait(0)

        g_start(j0, 0)

        @pl.when(t >= 1)
        def _():
            g_wait(1)
            sub(1)
            w_start(j0 - 1, 1)
            w_wait(1)

        g_start(j1, 1)
        g_wait(0)
        sub(0)
        w_start(j0, 0)
        return carry

    lax.fori_loop(0, _NCH // 2, body, 0)
    g_wait(1)
    sub(1)
    w_start(_NCH - 1, 1)
    w_wait(0)
    w_wait(1)


def _make_gather(width, tiled):
    mesh = plsc.VectorSubcoreMesh(core_axis_name="c", subcore_axis_name="s",
                                  num_cores=2)
    return pl.kernel(
        _sc_gather_body,
        out_type=(
            jax.ShapeDtypeStruct((_EP, width), jnp.float32),
            jax.ShapeDtypeStruct((_EP, width), jnp.float32),
        ),
        mesh=mesh,
        scratch_types=[
            pltpu.VMEM((_NCH, _CHUNK), jnp.int32),
            pltpu.VMEM((_NCH, _CHUNK), jnp.int32),
            pltpu.VMEM((_CHUNK, width), jnp.float32),
            pltpu.VMEM((_CHUNK, width), jnp.float32),
            pltpu.VMEM((_CHUNK, width), jnp.float32),
            pltpu.VMEM((_CHUNK, width), jnp.float32),
            pltpu.SemaphoreType.DMA,
            pltpu.SemaphoreType.DMA,
            pltpu.SemaphoreType.DMA,
            pltpu.SemaphoreType.DMA,
        ],
        compiler_params=pltpu.CompilerParams(use_tc_tiling_on_sc=tiled),
    )


def _make_scatter(width, tiled):
    mesh = plsc.VectorSubcoreMesh(core_axis_name="c", subcore_axis_name="s",
                                  num_cores=2)
    return pl.kernel(
        _sc_scatter_body,
        out_type=jax.ShapeDtypeStruct((2, _NP, width), jnp.float32),
        mesh=mesh,
        scratch_types=[
            pltpu.VMEM((_CHUNK,), jnp.int32),
            pltpu.VMEM((_CHUNK,), jnp.int32),
            pltpu.VMEM((_CHUNK, width), jnp.float32),
            pltpu.VMEM((_CHUNK, width), jnp.float32),
            pltpu.VMEM_SHARED((_NP, width), jnp.float32),
            pltpu.SemaphoreType.DMA,
            pltpu.SemaphoreType.DMA,
            pltpu.SemaphoreType.DMA,
            pltpu.SemaphoreType.DMA,
        ],
        compiler_params=pltpu.CompilerParams(use_tc_tiling_on_sc=tiled),
    )


def _make_gather_diff(width, tiled):
    mesh = plsc.VectorSubcoreMesh(core_axis_name="c", subcore_axis_name="s",
                                  num_cores=2)
    return pl.kernel(
        _sc_gather_diff_body,
        out_type=jax.ShapeDtypeStruct((_EP, width), jnp.float32),
        mesh=mesh,
        scratch_types=[
            pltpu.VMEM((_NCH, _CHUNK), jnp.int32),
            pltpu.VMEM((_NCH, _CHUNK), jnp.int32),
            pltpu.VMEM((_CHUNK, width), jnp.float32),
            pltpu.VMEM((_CHUNK, width), jnp.float32),
            pltpu.VMEM((_CHUNK, width), jnp.float32),
            pltpu.VMEM((_CHUNK, width), jnp.float32),
            pltpu.SemaphoreType.DMA,
            pltpu.SemaphoreType.DMA,
            pltpu.SemaphoreType.DMA,
            pltpu.SemaphoreType.DMA,
        ],
        compiler_params=pltpu.CompilerParams(use_tc_tiling_on_sc=tiled),
    )


@functools.cache
def _sc_kernels():
    return (_make_gather(128, True), _make_gather_diff(_XW, False),
            _make_scatter(128, True), _make_scatter(_XW, False))


def _sc_gather_big(a, bt, src3, dst3):
    return _sc_kernels()[0](a, bt, src3, dst3)


def _sc_gather_x(xp, src3, dst3):
    return _sc_kernels()[1](xp, xp, src3, dst3)


def _sc_scatter_big(s, dst3):
    return _sc_kernels()[2](s, dst3)


def _sc_scatter_x(sx, dst3):
    return _sc_kernels()[3](sx, dst3)


def _onehot16(ids_col, rows):
    return (ids_col == lax.broadcasted_iota(jnp.int32, (rows, 16), 1)
            ).astype(jnp.float32)


def _segsum0_body(eps_ref, gid_ref, out_ref):
    @pl.when(pl.program_id(0) == 0)
    def _():
        out_ref[...] = jnp.zeros_like(out_ref)

    oh = _onehot16(gid_ref[...], _BLKN)
    vals = jnp.concatenate(
        [eps_ref[:, :16], jnp.ones((_BLKN, 16), jnp.float32)], axis=1)
    out_ref[...] += lax.dot_general(
        oh, vals, (((0,), (0,)), ((), ())),
        preferred_element_type=jnp.float32)


def _segsum0(eps_p, gid_p):
    grid = _NP // _BLKN
    return pl.pallas_call(
        _segsum0_body,
        grid=(grid,),
        in_specs=[
            pl.BlockSpec((_BLKN, _XW), lambda i: (i, 0)),
            pl.BlockSpec((_BLKN, 1), lambda i: (i, 0)),
        ],
        out_specs=pl.BlockSpec((16, 32), lambda i: (0, 0)),
        out_shape=jax.ShapeDtypeStruct((16, 32), jnp.float32),
    )(eps_p, gid_p)


def _node_init_body(xyz_ref, eps_ref, aid_ref, gid_ref, gtab_ref, gmean_ref,
                    wemb_ref, wembt_ref, bemb_ref, h_ref, x_ref, epsc_ref):
    oh_g = _onehot16(gid_ref[...], _BLKN)
    per = jnp.dot(oh_g, gtab_ref[...], preferred_element_type=jnp.float32)
    meanp = jnp.dot(oh_g, gmean_ref[...], preferred_element_type=jnp.float32)
    alpha = per[:, 0:1]
    sigma = per[:, 1:2]
    tn = per[:, 2:3]
    epsc = eps_ref[...] - meanp
    x0 = alpha * xyz_ref[...] + sigma * epsc
    oh_a = _onehot16(aid_ref[...], _BLKN)
    h0 = _silu(jnp.dot(oh_a, wemb_ref[...], preferred_element_type=jnp.float32)
               + tn * wembt_ref[...] + bemb_ref[...])
    h_ref[...] = h0
    x_ref[...] = x0
    epsc_ref[...] = epsc


def _node_init(xyz_p, eps_p, aid_p, gid_p, gtab, gmean, wemb, wembt, bemb):
    grid = _NP // _BLKN
    return pl.pallas_call(
        _node_init_body,
        grid=(grid,),
        in_specs=[
            pl.BlockSpec((_BLKN, _XW), lambda i: (i, 0)),
            pl.BlockSpec((_BLKN, _XW), lambda i: (i, 0)),
            pl.BlockSpec((_BLKN, 1), lambda i: (i, 0)),
            pl.BlockSpec((_BLKN, 1), lambda i: (i, 0)),
            pl.BlockSpec((16, 8), lambda i: (0, 0)),
            pl.BlockSpec((16, _XW), lambda i: (0, 0)),
            pl.BlockSpec((16, 128), lambda i: (0, 0)),
            pl.BlockSpec((1, 128), lambda i: (0, 0)),
            pl.BlockSpec((1, 128), lambda i: (0, 0)),
        ],
        out_specs=[
            pl.BlockSpec((_BLKN, 128), lambda i: (i, 0)),
            pl.BlockSpec((_BLKN, _XW), lambda i: (i, 0)),
            pl.BlockSpec((_BLKN, _XW), lambda i: (i, 0)),
        ],
        out_shape=[
            jax.ShapeDtypeStruct((_NP, 128), jnp.float32),
            jax.ShapeDtypeStruct((_NP, _XW), jnp.float32),
            jax.ShapeDtypeStruct((_NP, _XW), jnp.float32),
        ],
    )(xyz_p, eps_p, aid_p, gid_p, gtab, gmean, wemb, wembt, bemb)


def _pre_body(h_ref, w1a_ref, w1b_ref, b1_ref, a_ref, bt_ref):
    h = h_ref[...]
    a_ref[...] = jnp.dot(h, w1a_ref[...],
                         preferred_element_type=jnp.float32) + b1_ref[...]
    bt_ref[...] = jnp.dot(h, w1b_ref[...],
                          preferred_element_type=jnp.float32)


def _pre(h, w1a, w1b, b1):
    grid = _NP // _BLKN
    return pl.pallas_call(
        _pre_body,
        grid=(grid,),
        in_specs=[
            pl.BlockSpec((_BLKN, 128), lambda i: (i, 0)),
            pl.BlockSpec((128, 128), lambda i: (0, 0)),
            pl.BlockSpec((128, 128), lambda i: (0, 0)),
            pl.BlockSpec((1, 128), lambda i: (0, 0)),
        ],
        out_specs=[
            pl.BlockSpec((_BLKN, 128), lambda i: (i, 0)),
            pl.BlockSpec((_BLKN, 128), lambda i: (i, 0)),
        ],
        out_shape=[
            jax.ShapeDtypeStruct((_NP, 128), jnp.float32),
            jax.ShapeDtypeStruct((_NP, 128), jnp.float32),
        ],
    )(h, w1a, w1b, b1)


def _edge_body(g1_ref, g2_ref, gxd_ref, w1c_ref, w2_ref, b2_ref,
               wx_ref, s_ref, sx_ref):
    pre = g1_ref[...] + g2_ref[...]
    diff = gxd_ref[...]
    d2 = jnp.sum(diff * diff, axis=1, keepdims=True)
    m1 = _silu(pre + d2 * w1c_ref[...])
    m2 = _silu(jnp.dot(m1, w2_ref[...], preferred_element_type=jnp.float32)
               + b2_ref[...])
    coef = jnp.tanh(jnp.sum(m2 * wx_ref[...], axis=1, keepdims=True))
    s_ref[...] = m2
    sx_ref[...] = diff * coef


def _edge(g1, g2, gxd, w1c, w2, b2, wx):
    grid = _EP // _BLKE
    return pl.pallas_call(
        _edge_body,
        grid=(grid,),
        in_specs=[
            pl.BlockSpec((_BLKE, 128), lambda i: (i, 0)),
            pl.BlockSpec((_BLKE, 128), lambda i: (i, 0)),
            pl.BlockSpec((_BLKE, _XW), lambda i: (i, 0)),
            pl.BlockSpec((1, 128), lambda i: (0, 0)),
            pl.BlockSpec((128, 128), lambda i: (0, 0)),
            pl.BlockSpec((1, 128), lambda i: (0, 0)),
            pl.BlockSpec((1, 128), lambda i: (0, 0)),
        ],
        out_specs=[
            pl.BlockSpec((_BLKE, 128), lambda i: (i, 0)),
            pl.BlockSpec((_BLKE, _XW), lambda i: (i, 0)),
        ],
        out_shape=[
            jax.ShapeDtypeStruct((_EP, 128), jnp.float32),
            jax.ShapeDtypeStruct((_EP, _XW), jnp.float32),
        ],
    )(g1, g2, gxd, w1c, w2, b2, wx)


def _update_body(h_ref, x_ref, p_ref, px_ref, wuh_ref, wua_ref, bu_ref,
                 h2_ref, x2_ref):
    agg = p_ref[0] + p_ref[1]
    aggx = px_ref[0] + px_ref[1]
    h = h_ref[...]
    z = (jnp.dot(h, wuh_ref[...], preferred_element_type=jnp.float32)
         + jnp.dot(agg, wua_ref[...], preferred_element_type=jnp.float32)
         + bu_ref[...])
    h2_ref[...] = h + _silu(z)
    x2_ref[...] = x_ref[...] + aggx * _XSCALE


def _update(h, x, p, px, wuh, wua, bu):
    grid = _NP // _BLKN
    return pl.pallas_call(
        _update_body,
        grid=(grid,),
        in_specs=[
            pl.BlockSpec((_BLKN, 128), lambda i: (i, 0)),
            pl.BlockSpec((_BLKN, _XW), lambda i: (i, 0)),
            pl.BlockSpec((2, _BLKN, 128), lambda i: (0, i, 0)),
            pl.BlockSpec((2, _BLKN, _XW), lambda i: (0, i, 0)),
            pl.BlockSpec((128, 128), lambda i: (0, 0)),
            pl.BlockSpec((128, 128), lambda i: (0, 0)),
            pl.BlockSpec((1, 128), lambda i: (0, 0)),
        ],
        out_specs=[
            pl.BlockSpec((_BLKN, 128), lambda i: (i, 0)),
            pl.BlockSpec((_BLKN, _XW), lambda i: (i, 0)),
        ],
        out_shape=[
            jax.ShapeDtypeStruct((_NP, 128), jnp.float32),
            jax.ShapeDtypeStruct((_NP, _XW), jnp.float32),
        ],
    )(h, x, p, px, wuh, wua, bu)


def _velsum_body(x3_ref, x0_ref, gid_ref, out_ref):
    @pl.when(pl.program_id(0) == 0)
    def _():
        out_ref[...] = jnp.zeros_like(out_ref)

    oh = _onehot16(gid_ref[...], _BLKN)
    vel = x3_ref[...] - x0_ref[...]
    out_ref[...] += lax.dot_general(
        oh, vel, (((0,), (0,)), ((), ())),
        preferred_element_type=jnp.float32)


def _velsum(x3, x0, gid_p):
    grid = _NP // _BLKN
    return pl.pallas_call(
        _velsum_body,
        grid=(grid,),
        in_specs=[
            pl.BlockSpec((_BLKN, _XW), lambda i: (i, 0)),
            pl.BlockSpec((_BLKN, _XW), lambda i: (i, 0)),
            pl.BlockSpec((_BLKN, 1), lambda i: (i, 0)),
        ],
        out_specs=pl.BlockSpec((16, _XW), lambda i: (0, 0)),
        out_shape=jax.ShapeDtypeStruct((16, _XW), jnp.float32),
    )(x3, x0, gid_p)


def _errsum_body(x3_ref, x0_ref, epsc_ref, gid_ref, mv_ref, out_ref):
    @pl.when(pl.program_id(0) == 0)
    def _():
        out_ref[...] = jnp.zeros_like(out_ref)

    oh = _onehot16(gid_ref[...], _BLKN)
    velc = (x3_ref[...] - x0_ref[...]
            - jnp.dot(oh, mv_ref[...], preferred_element_type=jnp.float32))
    err = (velc - epsc_ref[...]) ** 2
    out_ref[...] += lax.dot_general(
        oh, err, (((0,), (0,)), ((), ())),
        preferred_element_type=jnp.float32)


def _errsum(x3, x0, epsc, gid_p, mv):
    grid = _NP // _BLKN
    return pl.pallas_call(
        _errsum_body,
        grid=(grid,),
        in_specs=[
            pl.BlockSpec((_BLKN, _XW), lambda i: (i, 0)),
            pl.BlockSpec((_BLKN, _XW), lambda i: (i, 0)),
            pl.BlockSpec((_BLKN, _XW), lambda i: (i, 0)),
            pl.BlockSpec((_BLKN, 1), lambda i: (i, 0)),
            pl.BlockSpec((16, _XW), lambda i: (0, 0)),
        ],
        out_specs=pl.BlockSpec((16, _XW), lambda i: (0, 0)),
        out_shape=jax.ShapeDtypeStruct((16, _XW), jnp.float32),
    )(x3, x0, epsc, gid_p, mv)


def kernel(xyz, eps, atom_ids, edge_index, graph_ids, t_int,
           W_emb, b_emb, W_m1, b_m1, W_m2, b_m2, W_u, b_u, W_x):
    f32 = jnp.float32

    xn = t_int.astype(f32) / _T
    a2 = (1.0 - xn ** 2) ** 2
    a2 = (1.0 - 2.0 * _PREC) * a2 + _PREC
    gamma = jnp.log(1.0 - a2) - jnp.log(a2)
    alpha_g = jnp.sqrt(lax.logistic(-gamma))
    sigma_g = jnp.sqrt(lax.logistic(gamma))
    t_g = t_int.astype(f32) / _T

    xyz_p = jnp.zeros((_NP, _XW), f32).at[:_N, :3].set(xyz)
    eps_p = jnp.zeros((_NP, _XW), f32).at[:_N, :3].set(eps)
    aid_p = jnp.zeros((_NP, 1), jnp.int32).at[:_N, 0].set(
        atom_ids.astype(jnp.int32))
    gid_p = jnp.full((_NP, 1), _B, jnp.int32).at[:_N, 0].set(
        graph_ids.astype(jnp.int32))

    src = edge_index[0].astype(jnp.int32)
    dst = edge_index[1].astype(jnp.int32)
    src3 = jnp.zeros((_EP,), jnp.int32).at[:_E].set(src).reshape(
        _NWORK, _NCH, _CHUNK)
    dst3 = jnp.full((_EP,), _N, jnp.int32).at[:_E].set(dst).reshape(
        _NWORK, _NCH, _CHUNK)

    sums = _segsum0(eps_p, gid_p)
    cnt = sums[:, 16:17]
    mean_eps = jnp.zeros((16, _XW), f32).at[:, :16].set(sums[:, :16] / cnt)
    gtab = jnp.zeros((16, 8), f32)
    gtab = gtab.at[:, 0].set(alpha_g).at[:, 1].set(sigma_g).at[:, 2].set(t_g)

    h, x, epsc = _node_init(
        xyz_p, eps_p, aid_p, gid_p, gtab, mean_eps,
        W_emb[:_VOCAB], W_emb[_VOCAB:_VOCAB + 1], b_emb.reshape(1, 128))
    x0 = x

    for l in range(_NL):
        w1a = W_m1[l, :128]
        w1b = W_m1[l, 128:256]
        w1c = W_m1[l, 256:257]
        b1 = b_m1[l].reshape(1, 128)
        w2 = W_m2[l]
        b2 = b_m2[l].reshape(1, 128)
        wx = W_x[l].reshape(1, 128)
        wuh = W_u[l, :128]
        wua = W_u[l, 128:]
        bu = b_u[l].reshape(1, 128)

        a_tab, bt_tab = _pre(h, w1a, w1b, b1)
        g1, g2 = _sc_gather_big(a_tab, bt_tab, src3, dst3)
        gxd = _sc_gather_x(x, src3, dst3)
        s_msg, sx_msg = _edge(g1, g2, gxd, w1c, w2, b2, wx)
        p_acc = _sc_scatter_big(s_msg, dst3)
        px_acc = _sc_scatter_x(sx_msg, dst3)
        h, x = _update(h, x, p_acc, px_acc, wuh, wua, bu)

    vsums = _velsum(x, x0, gid_p)
    mv = vsums / cnt
    esums = _errsum(x, x0, epsc, gid_p, mv)
    per_graph = (esums[:, 0] + esums[:, 1] + esums[:, 2]) / (3.0 * cnt[:, 0])
    return per_graph


# speedup vs baseline: 1.3592x; 1.0175x over previous
import functools

import jax
import jax.numpy as jnp
from jax import lax
from jax.experimental import pallas as pl
from jax.experimental.pallas import tpu as pltpu
from jax.experimental.pallas import tpu_sc as plsc

_N = 10000
_E = 320000
_B = 16
_DH = 128
_NL = 3
_VOCAB = 16
_T = 1000
_PREC = 1e-05

_NP = 10240
_EP = 327680
_NWORK = 32
_CHUNK = 128
_NCH = _EP // (_NWORK * _CHUNK)
_XW = 16
_NR = _NP // 16
_BLKN = 1024
_BLKE = 4096
_XSCALE = 1.0 / (1.0 + _E / _N)


def _silu(z):
    return z * lax.logistic(z)


def _sc_gather_body(a_hbm, bt_hbm, src_hbm, dst_hbm, g1_hbm, g2_hbm,
                    idx_s, idx_d, b1_0, b1_1, b2_0, b2_1,
                    gsem0, gsem1, wsem0, wsem1):
    wid = lax.axis_index("s") * 2 + lax.axis_index("c")
    bufs1 = (b1_0, b1_1)
    bufs2 = (b2_0, b2_1)
    gsems = (gsem0, gsem1)
    wsems = (wsem0, wsem1)

    pltpu.sync_copy(src_hbm.at[wid], idx_s)
    pltpu.sync_copy(dst_hbm.at[wid], idx_d)

    def g_start(j, k):
        pltpu.async_copy(a_hbm.at[idx_s.at[j]], bufs1[k], gsems[k])
        pltpu.async_copy(bt_hbm.at[idx_d.at[j]], bufs2[k], gsems[k])

    def g_wait(k):
        pltpu.make_async_copy(a_hbm.at[idx_s.at[0]], bufs1[k], gsems[k]).wait()
        pltpu.make_async_copy(a_hbm.at[idx_s.at[0]], bufs2[k], gsems[k]).wait()

    def w_start(j, k):
        base = (wid * _NCH + j) * _CHUNK
        pltpu.async_copy(bufs1[k], g1_hbm.at[pl.ds(base, _CHUNK)], wsems[k])
        pltpu.async_copy(bufs2[k], g2_hbm.at[pl.ds(base, _CHUNK)], wsems[k])

    def w_wait(k):
        pltpu.make_async_copy(bufs1[k], g1_hbm.at[pl.ds(0, _CHUNK)],
                              wsems[k]).wait()
        pltpu.make_async_copy(bufs2[k], g2_hbm.at[pl.ds(0, _CHUNK)],
                              wsems[k]).wait()

    def body(t, carry):
        j0 = 2 * t
        j1 = j0 + 1

        @pl.when(t >= 1)
        def _():
            w_wait(0)

        g_start(j0, 0)

        @pl.when(t >= 1)
        def _():
            g_wait(1)
            w_start(j0 - 1, 1)
            w_wait(1)

        g_start(j1, 1)
        g_wait(0)
        w_start(j0, 0)
        return carry

    lax.fori_loop(0, _NCH // 2, body, 0)
    g_wait(1)
    w_start(_NCH - 1, 1)
    w_wait(0)
    w_wait(1)


def _sc_scatter_body(s_hbm, dst_hbm, p_hbm, idx_0, idx_1, b_0, b_1, acc,
                     lsem0, lsem1, csem0, csem1):
    cid = lax.axis_index("c")
    sid = lax.axis_index("s")
    wid = sid * 2 + cid
    idxs = (idx_0, idx_1)
    bufs = (b_0, b_1)
    lsems = (lsem0, lsem1)
    csems = (csem0, csem1)
    ncol = b_0.shape[1]

    def zrow(i, carry):
        for k in range(ncol // 16):
            b_0[i, pl.ds(k * 16, 16)] = jnp.zeros((16,), jnp.float32)
        return carry

    lax.fori_loop(0, _CHUNK, zrow, 0)

    def zacc(t, carry):
        pltpu.sync_copy(b_0, acc.at[pl.ds(sid * _NR + t * _CHUNK, _CHUNK)])
        return carry

    lax.fori_loop(0, _NR // _CHUNK, zacc, 0)
    plsc.subcore_barrier()

    def l_start(j, k):
        base = (wid * _NCH + j) * _CHUNK
        pltpu.async_copy(s_hbm.at[pl.ds(base, _CHUNK)], bufs[k], lsems[k])
        pltpu.async_copy(dst_hbm.at[wid, j], idxs[k], lsems[k])

    def l_wait(k):
        pltpu.make_async_copy(s_hbm.at[pl.ds(0, _CHUNK)], bufs[k],
                              lsems[k]).wait()
        pltpu.make_async_copy(dst_hbm.at[0, 0], idxs[k], lsems[k]).wait()

    def c_start(j, k):
        pltpu.async_copy(bufs[k], acc.at[idxs[k]], csems[k], add=True)

    def c_wait(k):
        pltpu.make_async_copy(bufs[k], acc.at[idxs[k]], csems[k]).wait()

    def body(t, carry):
        j0 = 2 * t
        j1 = j0 + 1

        @pl.when(t >= 1)
        def _():
            l_wait(1)
            c_start(j0 - 1, 1)
            c_wait(0)

        l_start(j0, 0)
        l_wait(0)

        @pl.when(t >= 1)
        def _():
            c_wait(1)

        c_start(j0, 0)
        l_start(j1, 1)
        return carry

    lax.fori_loop(0, _NCH // 2, body, 0)
    l_wait(1)
    c_start(_NCH - 1, 1)
    c_wait(0)
    c_wait(1)
    plsc.subcore_barrier()
    pltpu.sync_copy(acc.at[pl.ds(sid * _NR, _NR)],
                    p_hbm.at[cid, pl.ds(sid * _NR, _NR)])


def _sc_gather_diff_body(a_hbm, bt_hbm, src_hbm, dst_hbm, g1_hbm,
                         idx_s, idx_d, b1_0, b1_1, b2_0, b2_1,
                         gsem0, gsem1, wsem0, wsem1):
    wid = lax.axis_index("s") * 2 + lax.axis_index("c")
    bufs1 = (b1_0, b1_1)
    bufs2 = (b2_0, b2_1)
    gsems = (gsem0, gsem1)
    wsems = (wsem0, wsem1)

    pltpu.sync_copy(src_hbm.at[wid], idx_s)
    pltpu.sync_copy(dst_hbm.at[wid], idx_d)

    def g_start(j, k):
        pltpu.async_copy(a_hbm.at[idx_s.at[j]], bufs1[k], gsems[k])
        pltpu.async_copy(bt_hbm.at[idx_d.at[j]], bufs2[k], gsems[k])

    def g_wait(k):
        pltpu.make_async_copy(a_hbm.at[idx_s.at[0]], bufs1[k], gsems[k]).wait()
        pltpu.make_async_copy(a_hbm.at[idx_s.at[0]], bufs2[k], gsems[k]).wait()

    def sub(k):
        def row(i, carry):
            bufs1[k][i, :] = bufs1[k][i, :] - bufs2[k][i, :]
            return carry

        lax.fori_loop(0, _CHUNK, row, 0)

    def w_start(j, k):
        base = (wid * _NCH + j) * _CHUNK
        pltpu.async_copy(bufs1[k], g1_hbm.at[pl.ds(base, _CHUNK)], wsems[k])

    def w_wait(k):
        pltpu.make_async_copy(bufs1[k], g1_hbm.at[pl.ds(0, _CHUNK)],
                              wsems[k]).wait()

    def body(t, carry):
        j0 = 2 * t
        j1 = j0 + 1

        @pl.when(t >= 1)
        def _():
            w_wait(0)

        g_start(j0, 0)

        @pl.when(t >= 1)
        def _():
            g_wait(1)
            sub(1)
            w_start(j0 - 1, 1)
            w_wait(1)

        g_start(j1, 1)
        g_wait(0)
        sub(0)
        w_start(j0, 0)
        return carry

    lax.fori_loop(0, _NCH // 2, body, 0)
    g_wait(1)
    sub(1)
    w_start(_NCH - 1, 1)
    w_wait(0)
    w_wait(1)


def _make_gather(width, tiled):
    mesh = plsc.VectorSubcoreMesh(core_axis_name="c", subcore_axis_name="s",
                                  num_cores=2)
    return pl.kernel(
        _sc_gather_body,
        out_type=(
            jax.ShapeDtypeStruct((_EP, width), jnp.float32),
            jax.ShapeDtypeStruct((_EP, width), jnp.float32),
        ),
        mesh=mesh,
        scratch_types=[
            pltpu.VMEM((_NCH, _CHUNK), jnp.int32),
            pltpu.VMEM((_NCH, _CHUNK), jnp.int32),
            pltpu.VMEM((_CHUNK, width), jnp.float32),
            pltpu.VMEM((_CHUNK, width), jnp.float32),
            pltpu.VMEM((_CHUNK, width), jnp.float32),
            pltpu.VMEM((_CHUNK, width), jnp.float32),
            pltpu.SemaphoreType.DMA,
            pltpu.SemaphoreType.DMA,
            pltpu.SemaphoreType.DMA,
            pltpu.SemaphoreType.DMA,
        ],
        compiler_params=pltpu.CompilerParams(use_tc_tiling_on_sc=tiled),
    )


def _make_scatter(width, tiled):
    mesh = plsc.VectorSubcoreMesh(core_axis_name="c", subcore_axis_name="s",
                                  num_cores=2)
    return pl.kernel(
        _sc_scatter_body,
        out_type=jax.ShapeDtypeStruct((2, _NP, width), jnp.float32),
        mesh=mesh,
        scratch_types=[
            pltpu.VMEM((_CHUNK,), jnp.int32),
            pltpu.VMEM((_CHUNK,), jnp.int32),
            pltpu.VMEM((_CHUNK, width), jnp.float32),
            pltpu.VMEM((_CHUNK, width), jnp.float32),
            pltpu.VMEM_SHARED((_NP, width), jnp.float32),
            pltpu.SemaphoreType.DMA,
            pltpu.SemaphoreType.DMA,
            pltpu.SemaphoreType.DMA,
            pltpu.SemaphoreType.DMA,
        ],
        compiler_params=pltpu.CompilerParams(use_tc_tiling_on_sc=tiled),
    )


def _make_gather_diff(width, tiled):
    mesh = plsc.VectorSubcoreMesh(core_axis_name="c", subcore_axis_name="s",
                                  num_cores=2)
    return pl.kernel(
        _sc_gather_diff_body,
        out_type=jax.ShapeDtypeStruct((_EP, width), jnp.float32),
        mesh=mesh,
        scratch_types=[
            pltpu.VMEM((_NCH, _CHUNK), jnp.int32),
            pltpu.VMEM((_NCH, _CHUNK), jnp.int32),
            pltpu.VMEM((_CHUNK, width), jnp.float32),
            pltpu.VMEM((_CHUNK, width), jnp.float32),
            pltpu.VMEM((_CHUNK, width), jnp.float32),
            pltpu.VMEM((_CHUNK, width), jnp.float32),
            pltpu.SemaphoreType.DMA,
            pltpu.SemaphoreType.DMA,
            pltpu.SemaphoreType.DMA,
            pltpu.SemaphoreType.DMA,
        ],
        compiler_params=pltpu.CompilerParams(use_tc_tiling_on_sc=tiled),
    )


@functools.cache
def _sc_kernels():
    return (_make_gather(128, True), _make_gather_diff(_XW, False),
            _make_scatter(128, True), _make_scatter(_XW, False))


def _sc_gather_big(a, bt, src3, dst3):
    return _sc_kernels()[0](a, bt, src3, dst3)


def _sc_gather_x(xp, src3, dst3):
    return _sc_kernels()[1](xp, xp, src3, dst3)


def _sc_scatter_big(s, dst3):
    return _sc_kernels()[2](s, dst3)


def _sc_scatter_x(sx, dst3):
    return _sc_kernels()[3](sx, dst3)


def _onehot16(ids_col, rows):
    return (ids_col == lax.broadcasted_iota(jnp.int32, (rows, 16), 1)
            ).astype(jnp.float32)


def _segsum0_body(eps_ref, gid_ref, out_ref):
    @pl.when(pl.program_id(0) == 0)
    def _():
        out_ref[...] = jnp.zeros_like(out_ref)

    oh = _onehot16(gid_ref[...], _BLKN)
    vals = jnp.concatenate(
        [eps_ref[:, :16], jnp.ones((_BLKN, 16), jnp.float32)], axis=1)
    out_ref[...] += lax.dot_general(
        oh, vals, (((0,), (0,)), ((), ())),
        preferred_element_type=jnp.float32)


def _segsum0(eps_p, gid_p):
    grid = _NP // _BLKN
    return pl.pallas_call(
        _segsum0_body,
        grid=(grid,),
        in_specs=[
            pl.BlockSpec((_BLKN, _XW), lambda i: (i, 0)),
            pl.BlockSpec((_BLKN, 1), lambda i: (i, 0)),
        ],
        out_specs=pl.BlockSpec((16, 32), lambda i: (0, 0)),
        out_shape=jax.ShapeDtypeStruct((16, 32), jnp.float32),
    )(eps_p, gid_p)


def _node_init_body(xyz_ref, eps_ref, aid_ref, gid_ref, gtab_ref, gmean_ref,
                    wemb_ref, wembt_ref, bemb_ref, h_ref, x_ref, epsc_ref):
    oh_g = _onehot16(gid_ref[...], _BLKN)
    per = jnp.dot(oh_g, gtab_ref[...], preferred_element_type=jnp.float32)
    meanp = jnp.dot(oh_g, gmean_ref[...], preferred_element_type=jnp.float32)
    alpha = per[:, 0:1]
    sigma = per[:, 1:2]
    tn = per[:, 2:3]
    epsc = eps_ref[...] - meanp
    x0 = alpha * xyz_ref[...] + sigma * epsc
    oh_a = _onehot16(aid_ref[...], _BLKN)
    h0 = _silu(jnp.dot(oh_a, wemb_ref[...], preferred_element_type=jnp.float32)
               + tn * wembt_ref[...] + bemb_ref[...])
    h_ref[...] = h0
    x_ref[...] = x0
    epsc_ref[...] = epsc


def _node_init(xyz_p, eps_p, aid_p, gid_p, gtab, gmean, wemb, wembt, bemb):
    grid = _NP // _BLKN
    return pl.pallas_call(
        _node_init_body,
        grid=(grid,),
        in_specs=[
            pl.BlockSpec((_BLKN, _XW), lambda i: (i, 0)),
            pl.BlockSpec((_BLKN, _XW), lambda i: (i, 0)),
            pl.BlockSpec((_BLKN, 1), lambda i: (i, 0)),
            pl.BlockSpec((_BLKN, 1), lambda i: (i, 0)),
            pl.BlockSpec((16, 8), lambda i: (0, 0)),
            pl.BlockSpec((16, _XW), lambda i: (0, 0)),
            pl.BlockSpec((16, 128), lambda i: (0, 0)),
            pl.BlockSpec((1, 128), lambda i: (0, 0)),
            pl.BlockSpec((1, 128), lambda i: (0, 0)),
        ],
        out_specs=[
            pl.BlockSpec((_BLKN, 128), lambda i: (i, 0)),
            pl.BlockSpec((_BLKN, _XW), lambda i: (i, 0)),
            pl.BlockSpec((_BLKN, _XW), lambda i: (i, 0)),
        ],
        out_shape=[
            jax.ShapeDtypeStruct((_NP, 128), jnp.float32),
            jax.ShapeDtypeStruct((_NP, _XW), jnp.float32),
            jax.ShapeDtypeStruct((_NP, _XW), jnp.float32),
        ],
    )(xyz_p, eps_p, aid_p, gid_p, gtab, gmean, wemb, wembt, bemb)


def _pre_body(h_ref, w1a_ref, w1b_ref, b1_ref, a_ref, bt_ref):
    h = h_ref[...]
    a_ref[...] = jnp.dot(h, w1a_ref[...],
                         preferred_element_type=jnp.float32) + b1_ref[...]
    bt_ref[...] = jnp.dot(h, w1b_ref[...],
                          preferred_element_type=jnp.float32)


def _pre(h, w1a, w1b, b1):
    grid = _NP // _BLKN
    return pl.pallas_call(
        _pre_body,
        grid=(grid,),
        in_specs=[
            pl.BlockSpec((_BLKN, 128), lambda i: (i, 0)),
            pl.BlockSpec((128, 128), lambda i: (0, 0)),
            pl.BlockSpec((128, 128), lambda i: (0, 0)),
            pl.BlockSpec((1, 128), lambda i: (0, 0)),
        ],
        out_specs=[
            pl.BlockSpec((_BLKN, 128), lambda i: (i, 0)),
            pl.BlockSpec((_BLKN, 128), lambda i: (i, 0)),
        ],
        out_shape=[
            jax.ShapeDtypeStruct((_NP, 128), jnp.float32),
            jax.ShapeDtypeStruct((_NP, 128), jnp.float32),
        ],
    )(h, w1a, w1b, b1)


def _edge_body(g1_ref, g2_ref, gxd_ref, w1c_ref, w2_ref, b2_ref,
               wx_ref, s_ref, sx_ref):
    pre = g1_ref[...] + g2_ref[...]
    diff = gxd_ref[...]
    d2 = jnp.sum(diff * diff, axis=1, keepdims=True)
    m1 = _silu(pre + d2 * w1c_ref[...])
    m2 = _silu(jnp.dot(m1, w2_ref[...], preferred_element_type=jnp.float32)
               + b2_ref[...])
    coef = jnp.tanh(jnp.sum(m2 * wx_ref[...], axis=1, keepdims=True))
    s_ref[...] = m2
    sx_ref[...] = diff * coef


def _edge(g1, g2, gxd, w1c, w2, b2, wx):
    grid = _EP // _BLKE
    return pl.pallas_call(
        _edge_body,
        grid=(grid,),
        in_specs=[
            pl.BlockSpec((_BLKE, 128), lambda i: (i, 0)),
            pl.BlockSpec((_BLKE, 128), lambda i: (i, 0)),
            pl.BlockSpec((_BLKE, _XW), lambda i: (i, 0)),
            pl.BlockSpec((1, 128), lambda i: (0, 0)),
            pl.BlockSpec((128, 128), lambda i: (0, 0)),
            pl.BlockSpec((1, 128), lambda i: (0, 0)),
            pl.BlockSpec((1, 128), lambda i: (0, 0)),
        ],
        out_specs=[
            pl.BlockSpec((_BLKE, 128), lambda i: (i, 0)),
            pl.BlockSpec((_BLKE, _XW), lambda i: (i, 0)),
        ],
        out_shape=[
            jax.ShapeDtypeStruct((_EP, 128), jnp.float32),
            jax.ShapeDtypeStruct((_EP, _XW), jnp.float32),
        ],
    )(g1, g2, gxd, w1c, w2, b2, wx)


def _update_body(h_ref, x_ref, p_ref, px_ref, wuh_ref, wua_ref, bu_ref,
                 h2_ref, x2_ref):
    agg = p_ref[0] + p_ref[1]
    aggx = px_ref[0] + px_ref[1]
    h = h_ref[...]
    z = (jnp.dot(h, wuh_ref[...], preferred_element_type=jnp.float32)
         + jnp.dot(agg, wua_ref[...], preferred_element_type=jnp.float32)
         + bu_ref[...])
    h2_ref[...] = h + _silu(z)
    x2_ref[...] = x_ref[...] + aggx * _XSCALE


def _update(h, x, p, px, wuh, wua, bu):
    grid = _NP // _BLKN
    return pl.pallas_call(
        _update_body,
        grid=(grid,),
        in_specs=[
            pl.BlockSpec((_BLKN, 128), lambda i: (i, 0)),
            pl.BlockSpec((_BLKN, _XW), lambda i: (i, 0)),
            pl.BlockSpec((2, _BLKN, 128), lambda i: (0, i, 0)),
            pl.BlockSpec((2, _BLKN, _XW), lambda i: (0, i, 0)),
            pl.BlockSpec((128, 128), lambda i: (0, 0)),
            pl.BlockSpec((128, 128), lambda i: (0, 0)),
            pl.BlockSpec((1, 128), lambda i: (0, 0)),
        ],
        out_specs=[
            pl.BlockSpec((_BLKN, 128), lambda i: (i, 0)),
            pl.BlockSpec((_BLKN, _XW), lambda i: (i, 0)),
        ],
        out_shape=[
            jax.ShapeDtypeStruct((_NP, 128), jnp.float32),
            jax.ShapeDtypeStruct((_NP, _XW), jnp.float32),
        ],
    )(h, x, p, px, wuh, wua, bu)


def _velsum_body(x3_ref, x0_ref, gid_ref, out_ref):
    @pl.when(pl.program_id(0) == 0)
    def _():
        out_ref[...] = jnp.zeros_like(out_ref)

    oh = _onehot16(gid_ref[...], _BLKN)
    vel = x3_ref[...] - x0_ref[...]
    out_ref[...] += lax.dot_general(
        oh, vel, (((0,), (0,)), ((), ())),
        preferred_element_type=jnp.float32)


def _velsum(x3, x0, gid_p):
    grid = _NP // _BLKN
    return pl.pallas_call(
        _velsum_body,
        grid=(grid,),
        in_specs=[
            pl.BlockSpec((_BLKN, _XW), lambda i: (i, 0)),
            pl.BlockSpec((_BLKN, _XW), lambda i: (i, 0)),
            pl.BlockSpec((_BLKN, 1), lambda i: (i, 0)),
        ],
        out_specs=pl.BlockSpec((16, _XW), lambda i: (0, 0)),
        out_shape=jax.ShapeDtypeStruct((16, _XW), jnp.float32),
    )(x3, x0, gid_p)


def _errsum_body(x3_ref, x0_ref, epsc_ref, gid_ref, mv_ref, out_ref):
    @pl.when(pl.program_id(0) == 0)
    def _():
        out_ref[...] = jnp.zeros_like(out_ref)

    oh = _onehot16(gid_ref[...], _BLKN)
    velc = (x3_ref[...] - x0_ref[...]
            - jnp.dot(oh, mv_ref[...], preferred_element_type=jnp.float32))
    err = (velc - epsc_ref[...]) ** 2
    out_ref[...] += lax.dot_general(
        oh, err, (((0,), (0,)), ((), ())),
        preferred_element_type=jnp.float32)


def _errsum(x3, x0, epsc, gid_p, mv):
    grid = _NP // _BLKN
    return pl.pallas_call(
        _errsum_body,
        grid=(grid,),
        in_specs=[
            pl.BlockSpec((_BLKN, _XW), lambda i: (i, 0)),
            pl.BlockSpec((_BLKN, _XW), lambda i: (i, 0)),
            pl.BlockSpec((_BLKN, _XW), lambda i: (i, 0)),
            pl.BlockSpec((_BLKN, 1), lambda i: (i, 0)),
            pl.BlockSpec((16, _XW), lambda i: (0, 0)),
        ],
        out_specs=pl.BlockSpec((16, _XW), lambda i: (0, 0)),
        out_shape=jax.ShapeDtypeStruct((16, _XW), jnp.float32),
    )(x3, x0, epsc, gid_p, mv)


def kernel(xyz, eps, atom_ids, edge_index, graph_ids, t_int,
           W_emb, b_emb, W_m1, b_m1, W_m2, b_m2, W_u, b_u, W_x):
    f32 = jnp.float32

    xn = t_int.astype(f32) / _T
    a2 = (1.0 - xn ** 2) ** 2
    a2 = (1.0 - 2.0 * _PREC) * a2 + _PREC
    gamma = jnp.log(1.0 - a2) - jnp.log(a2)
    alpha_g = jnp.sqrt(lax.logistic(-gamma))
    sigma_g = jnp.sqrt(lax.logistic(gamma))
    t_g = t_int.astype(f32) / _T

    xyz_p = jnp.zeros((_NP, _XW), f32).at[:_N, :3].set(xyz)
    eps_p = jnp.zeros((_NP, _XW), f32).at[:_N, :3].set(eps)
    aid_p = jnp.zeros((_NP, 1), jnp.int32).at[:_N, 0].set(
        atom_ids.astype(jnp.int32))
    gid_p = jnp.full((_NP, 1), _B, jnp.int32).at[:_N, 0].set(
        graph_ids.astype(jnp.int32))

    src = edge_index[0].astype(jnp.int32)
    dst = edge_index[1].astype(jnp.int32)
    src3 = jnp.zeros((_EP,), jnp.int32).at[:_E].set(src).reshape(
        _NWORK, _NCH, _CHUNK)
    dst3 = jnp.full((_EP,), _N, jnp.int32).at[:_E].set(dst).reshape(
        _NWORK, _NCH, _CHUNK)

    sums = _segsum0(eps_p, gid_p)
    cnt = sums[:, 16:17]
    mean_eps = jnp.zeros((16, _XW), f32).at[:, :16].set(sums[:, :16] / cnt)
    gtab = jnp.zeros((16, 8), f32)
    gtab = gtab.at[:, 0].set(alpha_g).at[:, 1].set(sigma_g).at[:, 2].set(t_g)

    h, x, epsc = _node_init(
        xyz_p, eps_p, aid_p, gid_p, gtab, mean_eps,
        W_emb[:_VOCAB], W_emb[_VOCAB:_VOCAB + 1], b_emb.reshape(1, 128))
    x0 = x

    for l in range(_NL):
        w1a = W_m1[l, :128]
        w1b = W_m1[l, 128:256]
        w1c = W_m1[l, 256:257]
        b1 = b_m1[l].reshape(1, 128)
        w2 = W_m2[l]
        b2 = b_m2[l].reshape(1, 128)
        wx = W_x[l].reshape(1, 128)
        wuh = W_u[l, :128]
        wua = W_u[l, 128:]
        bu = b_u[l].reshape(1, 128)

        a_tab, bt_tab = _pre(h, w1a, w1b, b1)
        g1, g2 = _sc_gather_big(a_tab, bt_tab, src3, dst3)
        gxd = _sc_gather_x(x, src3, dst3)
        s_msg, sx_msg = _edge(g1, g2, gxd, w1c, w2, b2, wx)
        p_acc = _sc_scatter_big(s_msg, dst3)
        px_acc = _sc_scatter_x(sx_msg, dst3)
        h, x = _update(h, x, p_acc, px_acc, wuh, wua, bu)

    vsums = _velsum(x, x0, gid_p)
    mv = vsums / cnt
    esums = _errsum(x, x0, epsc, gid_p, mv)
    per_graph = (esums[:, 0] + esums[:, 1] + esums[:, 2]) / (3.0 * cnt[:, 0])
    return per_graph
